# baseline (speedup 1.0000x reference)
import jax
import jax.numpy as jnp
from jax import lax
from jax.experimental import pallas as pl
from jax.experimental.pallas import tpu as pltpu
from jax.experimental.pallas import tpu_sc as plsc

B, L, DIM = 4096, 50, 64
LP = 56
NP = B * LP
NC, NS = 2, 16
NW = NC * NS
PER_W = NP // NW
TCH = 64
ICH = 2 * TCH
NCH = PER_W // TCH
FROWS = 100001
GROWS = FROWS + 1001


def _sc_gather_body(ids_hbm, gtab_hbm, x_hbm, idx_v, xbuf, sems):
    wid = lax.axis_index("s") * NC + lax.axis_index("c")
    ibase = wid * PER_W * 2
    pltpu.sync_copy(ids_hbm.at[pl.ds(ibase, 2 * PER_W)], idx_v)

    def gather(j, slot):
        jc = lax.min(j, NCH - 1)
        pltpu.async_copy(gtab_hbm.at[idx_v.at[pl.ds(jc * ICH, ICH)]],
                         xbuf.at[slot], sems.at[slot])

    def wait_write(j, slot):
        pltpu.make_async_copy(gtab_hbm.at[idx_v.at[pl.ds(0, ICH)]],
                              xbuf.at[slot], sems.at[slot]).wait()
        pltpu.sync_copy(xbuf.at[slot], x_hbm.at[pl.ds(ibase + j * ICH, ICH)])

    gather(0, 0)
    gather(1, 1)

    def pair(jj, carry):
        j0 = jj * 2
        wait_write(j0, 0)
        gather(j0 + 2, 0)
        wait_write(j0 + 1, 1)
        gather(j0 + 3, 1)
        return carry

    lax.fori_loop(0, NCH // 2, pair, 0)
    for slot in (0, 1):
        pltpu.make_async_copy(gtab_hbm.at[idx_v.at[pl.ds(0, ICH)]],
                              xbuf.at[slot], sems.at[slot]).wait()


def _sc_gather(ids2, gtab):
    return pl.kernel(
        _sc_gather_body,
        out_type=jax.ShapeDtypeStruct((2 * NP, DIM), jnp.float32),
        mesh=plsc.VectorSubcoreMesh(core_axis_name="c", subcore_axis_name="s",
                                    num_cores=NC, num_subcores=NS),
        scratch_types=[
            pltpu.VMEM((2 * PER_W,), jnp.int32),
            pltpu.VMEM((2, ICH, DIM), jnp.float32),
            pltpu.SemaphoreType.DMA((2,)),
        ],
        compiler_params=pltpu.CompilerParams(use_tc_tiling_on_sc=False),
    )(ids2, gtab)


BB = 64


def _tc_mlp_body(x_ref, w1_ref, b1_ref, w2_ref, b2_ref, o_ref):
    x = x_ref[...]
    h = jnp.dot(x, w1_ref[...], preferred_element_type=jnp.float32)
    h = jnp.maximum(h + b1_ref[...], 0.0)
    y = (jnp.dot(h, w2_ref[...], preferred_element_type=jnp.float32)
         + b2_ref[...])
    for b in range(BB):
        o_ref[b] = y[b * LP:b * LP + L, :]


def _tc_mlp(x2, W1, b1, W2, b2, *, interpret=False):
    return pl.pallas_call(
        _tc_mlp_body,
        grid=(B // BB,),
        in_specs=[
            pl.BlockSpec((BB * LP, 2 * DIM), lambda i: (i, 0)),
            pl.BlockSpec((2 * DIM, 2 * DIM), lambda i: (0, 0)),
            pl.BlockSpec((1, 2 * DIM), lambda i: (0, 0)),
            pl.BlockSpec((2 * DIM, DIM), lambda i: (0, 0)),
            pl.BlockSpec((1, DIM), lambda i: (0, 0)),
        ],
        out_specs=pl.BlockSpec((BB, L, DIM), lambda i: (i, 0, 0)),
        out_shape=jax.ShapeDtypeStruct((B, L, DIM), jnp.float32),
        interpret=interpret,
    )(x2, W1, b1, W2, b2)


def kernel(fine_ids, coarse_ids, fine_table, coarse_table, W1, b1, W2, b2):
    fill = jnp.arange(B * (LP - L), dtype=jnp.int32).reshape(B, LP - L)
    fpad = jnp.concatenate([fine_ids.astype(jnp.int32),
                            (fill * 997) % FROWS], axis=1)
    cpad = jnp.concatenate([coarse_ids.astype(jnp.int32),
                            fill % 1001], axis=1)
    fi = fpad.reshape(-1)
    ci = cpad.reshape(-1) + FROWS
    ids2 = jnp.stack([fi, ci], axis=-1).reshape(-1)
    gtab = jnp.concatenate([fine_table, coarse_table], axis=0)
    xrows = _sc_gather(ids2, gtab)
    x2 = xrows.reshape(NP, 2 * DIM)
    out = _tc_mlp(x2, W1, b1.reshape(1, 2 * DIM), W2, b2.reshape(1, DIM))
    return out, jnp.float32(0.5)


# speedup vs baseline: 1.8803x; 1.8803x over previous
import jax
import jax.numpy as jnp
from jax import lax
from jax.experimental import pallas as pl
from jax.experimental.pallas import tpu as pltpu
from jax.experimental.pallas import tpu_sc as plsc

B, L, DIM = 4096, 50, 64
NL = B * L
NC, NS = 2, 16
NW = NC * NS
PER_W = NL // NW
TCH = 64
ICH = 2 * TCH
NCH = PER_W // TCH
NBUF = 4
FROWS = 100001
GROWS = FROWS + 1001


def _sc_gather_body(fidx_hbm, cidx_hbm, gtab_hbm, x_hbm,
                    ftmp, ctmp, ibuf, xbuf, sems):
    wid = lax.axis_index("s") * NC + lax.axis_index("c")
    tbase = wid * PER_W
    pltpu.sync_copy(fidx_hbm.at[pl.ds(tbase, PER_W)], ftmp)
    pltpu.sync_copy(cidx_hbm.at[pl.ds(tbase, PER_W)], ctmp)

    lanes = lax.iota(jnp.int32, 16) * 2

    def ileave(g, carry):
        fv = ftmp[pl.ds(g * 16, 16)]
        cv = ctmp[pl.ds(g * 16, 16)]
        plsc.store_scatter(ibuf, [lanes + g * 32], fv)
        plsc.store_scatter(ibuf, [lanes + g * 32 + 1], cv)
        return carry

    lax.fori_loop(0, PER_W // 16, ileave, 0)

    def gather(j, slot):
        jc = lax.min(j, NCH - 1)
        pltpu.async_copy(gtab_hbm.at[ibuf.at[pl.ds(jc * ICH, ICH)]],
                         xbuf.at[slot], sems.at[slot])

    def wait_write(j, slot):
        pltpu.make_async_copy(gtab_hbm.at[ibuf.at[pl.ds(0, ICH)]],
                              xbuf.at[slot], sems.at[slot]).wait()
        pltpu.sync_copy(xbuf.at[slot],
                        x_hbm.at[pl.ds((tbase + j * TCH) * 2, ICH)])

    for s in range(NBUF):
        gather(s, s)

    def group(jj, carry):
        j0 = jj * NBUF
        for s in range(NBUF):
            wait_write(j0 + s, s)
            gather(j0 + s + NBUF, s)
        return carry

    lax.fori_loop(0, NCH // NBUF, group, 0)
    for slot in range(NBUF):
        pltpu.make_async_copy(gtab_hbm.at[ibuf.at[pl.ds(0, ICH)]],
                              xbuf.at[slot], sems.at[slot]).wait()


def _sc_gather(fidx, cidx, gtab):
    return pl.kernel(
        _sc_gather_body,
        out_type=jax.ShapeDtypeStruct((2 * NL, DIM), jnp.float32),
        mesh=plsc.VectorSubcoreMesh(core_axis_name="c", subcore_axis_name="s",
                                    num_cores=NC, num_subcores=NS),
        scratch_types=[
            pltpu.VMEM((PER_W,), jnp.int32),
            pltpu.VMEM((PER_W,), jnp.int32),
            pltpu.VMEM((2 * PER_W,), jnp.int32),
            pltpu.VMEM((NBUF, ICH, DIM), jnp.float32),
            pltpu.SemaphoreType.DMA((NBUF,)),
        ],
        compiler_params=pltpu.CompilerParams(use_tc_tiling_on_sc=False,
                                             needs_layout_passes=False),
    )(fidx, cidx, gtab)


def _tc_mlp_body(x_ref, w1_ref, b1_ref, w2_ref, b2_ref, o_ref):
    x = x_ref[...]
    h = jnp.dot(x, w1_ref[...], preferred_element_type=jnp.float32)
    h = jnp.maximum(h + b1_ref[...], 0.0)
    y = (jnp.dot(h, w2_ref[...], preferred_element_type=jnp.float32)
         + b2_ref[...])
    o_ref[0] = jnp.transpose(y)


def _tc_mlp(x2, W1, b1, W2, b2, *, interpret=False):
    return pl.pallas_call(
        _tc_mlp_body,
        grid=(L,),
        in_specs=[
            pl.BlockSpec((B, 2 * DIM), lambda i: (i, 0)),
            pl.BlockSpec((2 * DIM, 2 * DIM), lambda i: (0, 0)),
            pl.BlockSpec((1, 2 * DIM), lambda i: (0, 0)),
            pl.BlockSpec((2 * DIM, DIM), lambda i: (0, 0)),
            pl.BlockSpec((1, DIM), lambda i: (0, 0)),
        ],
        out_specs=pl.BlockSpec((1, DIM, B), lambda i: (i, 0, 0)),
        out_shape=jax.ShapeDtypeStruct((L, DIM, B), jnp.float32),
        interpret=interpret,
    )(x2, W1, b1, W2, b2)


def kernel(fine_ids, coarse_ids, fine_table, coarse_table, W1, b1, W2, b2):
    fiT = fine_ids.astype(jnp.int32).T.reshape(NL)
    ciT = coarse_ids.astype(jnp.int32).T.reshape(NL) + FROWS
    gtab = jnp.concatenate([fine_table.reshape(FROWS * DIM),
                            coarse_table.reshape(1001 * DIM)])
    xrows = _sc_gather(fiT, ciT, gtab.reshape(GROWS, DIM))
    x2 = xrows.reshape(NL, 2 * DIM)
    outT = _tc_mlp(x2, W1, b1.reshape(1, 2 * DIM), W2, b2.reshape(1, DIM))
    return jnp.transpose(outT, (2, 0, 1)), jnp.float32(0.5)


# speedup vs baseline: 2.0297x; 1.0795x over previous
import jax
import jax.numpy as jnp
from jax import lax
from jax.experimental import pallas as pl
from jax.experimental.pallas import tpu as pltpu
from jax.experimental.pallas import tpu_sc as plsc

B, L, DIM = 4096, 50, 64
NL = B * L
NC, NS = 2, 16
NW = NC * NS
PER_W = NL // NW
TCH = 128
NCH = PER_W // TCH
NBUF = 2
FROWS = 100001


def _sc_gather_body(fidx_hbm, cidx_hbm, ftab_hbm, ctab_hbm, x_hbm,
                    ftmp, ctmp, fbuf, cbuf, fsems, csems):
    wid = lax.axis_index("s") * NC + lax.axis_index("c")
    tbase = wid * PER_W
    pltpu.sync_copy(fidx_hbm.at[pl.ds(tbase, PER_W)], ftmp)
    pltpu.sync_copy(cidx_hbm.at[pl.ds(tbase, PER_W)], ctmp)

    def gather(j, slot):
        jc = lax.min(j, NCH - 1)
        pltpu.async_copy(ftab_hbm.at[ftmp.at[pl.ds(jc * TCH, TCH)]],
                         fbuf.at[slot], fsems.at[slot])
        pltpu.async_copy(ctab_hbm.at[ctmp.at[pl.ds(jc * TCH, TCH)]],
                         cbuf.at[slot], csems.at[slot])

    def wait_write(j, slot):
        pltpu.make_async_copy(ftab_hbm.at[ftmp.at[pl.ds(0, TCH)]],
                              fbuf.at[slot], fsems.at[slot]).wait()
        pltpu.make_async_copy(ctab_hbm.at[ctmp.at[pl.ds(0, TCH)]],
                              cbuf.at[slot], csems.at[slot]).wait()
        row0 = tbase + j * TCH
        pltpu.sync_copy(fbuf.at[slot],
                        x_hbm.at[pl.ds(row0, TCH), pl.ds(0, DIM)])
        pltpu.sync_copy(cbuf.at[slot],
                        x_hbm.at[pl.ds(row0, TCH), pl.ds(DIM, DIM)])

    for s in range(NBUF):
        gather(s, s)

    def group(jj, carry):
        j0 = jj * NBUF
        for s in range(NBUF):
            wait_write(j0 + s, s)
            gather(j0 + s + NBUF, s)
        return carry

    lax.fori_loop(0, NCH // NBUF, group, 0)
    for slot in range(NBUF):
        pltpu.make_async_copy(ftab_hbm.at[ftmp.at[pl.ds(0, TCH)]],
                              fbuf.at[slot], fsems.at[slot]).wait()
        pltpu.make_async_copy(ctab_hbm.at[ctmp.at[pl.ds(0, TCH)]],
                              cbuf.at[slot], csems.at[slot]).wait()


def _sc_gather(fidx, cidx, ftab, ctab):
    return pl.kernel(
        _sc_gather_body,
        out_type=jax.ShapeDtypeStruct((NL, 2 * DIM), jnp.float32),
        mesh=plsc.VectorSubcoreMesh(core_axis_name="c", subcore_axis_name="s",
                                    num_cores=NC, num_subcores=NS),
        scratch_types=[
            pltpu.VMEM((PER_W,), jnp.int32),
            pltpu.VMEM((PER_W,), jnp.int32),
            pltpu.VMEM((NBUF, TCH, DIM), jnp.float32),
            pltpu.VMEM((NBUF, TCH, DIM), jnp.float32),
            pltpu.SemaphoreType.DMA((NBUF,)),
            pltpu.SemaphoreType.DMA((NBUF,)),
        ],
        compiler_params=pltpu.CompilerParams(use_tc_tiling_on_sc=False),
    )(fidx, cidx, ftab, ctab)


def _tc_mlp_body(x_ref, w1_ref, b1_ref, w2_ref, b2_ref, o_ref):
    x = x_ref[...]
    h = jnp.dot(x, w1_ref[...], preferred_element_type=jnp.float32)
    h = jnp.maximum(h + b1_ref[...], 0.0)
    y = (jnp.dot(h, w2_ref[...], preferred_element_type=jnp.float32)
         + b2_ref[...])
    o_ref[0] = jnp.transpose(y)


def _tc_mlp(x2, W1, b1, W2, b2, *, interpret=False):
    return pl.pallas_call(
        _tc_mlp_body,
        grid=(L,),
        in_specs=[
            pl.BlockSpec((B, 2 * DIM), lambda i: (i, 0)),
            pl.BlockSpec((2 * DIM, 2 * DIM), lambda i: (0, 0)),
            pl.BlockSpec((1, 2 * DIM), lambda i: (0, 0)),
            pl.BlockSpec((2 * DIM, DIM), lambda i: (0, 0)),
            pl.BlockSpec((1, DIM), lambda i: (0, 0)),
        ],
        out_specs=pl.BlockSpec((1, DIM, B), lambda i: (i, 0, 0)),
        out_shape=jax.ShapeDtypeStruct((L, DIM, B), jnp.float32),
        interpret=interpret,
    )(x2, W1, b1, W2, b2)


def kernel(fine_ids, coarse_ids, fine_table, coarse_table, W1, b1, W2, b2):
    fiT = fine_ids.astype(jnp.int32).T.reshape(NL)
    ciT = coarse_ids.astype(jnp.int32).T.reshape(NL)
    x2 = _sc_gather(fiT, ciT, fine_table, coarse_table)
    outT = _tc_mlp(x2, W1, b1.reshape(1, 2 * DIM), W2, b2.reshape(1, DIM))
    return jnp.transpose(outT, (2, 0, 1)), jnp.float32(0.5)


# speedup vs baseline: 2.1911x; 1.0795x over previous
import jax
import jax.numpy as jnp
from jax import lax
from jax.experimental import pallas as pl
from jax.experimental.pallas import tpu as pltpu
from jax.experimental.pallas import tpu_sc as plsc

B, L, DIM = 4096, 50, 64
NL = B * L
NC, NS = 2, 16
NW = NC * NS
PER_W = NL // NW
TCH = 128
NCH = PER_W // TCH
NBUF = 5
FROWS = 100001


def _sc_gather_body(fidx_hbm, cidx_hbm, ftab_hbm, ctab_hbm, x_hbm,
                    ftmp, ctmp, fbuf, cbuf, fsems, csems):
    wid = lax.axis_index("s") * NC + lax.axis_index("c")
    tbase = wid * PER_W
    pltpu.sync_copy(fidx_hbm.at[pl.ds(tbase, PER_W)], ftmp)
    pltpu.sync_copy(cidx_hbm.at[pl.ds(tbase, PER_W)], ctmp)

    def gather(j, slot):
        jc = lax.min(j, NCH - 1)
        pltpu.async_copy(ftab_hbm.at[ftmp.at[pl.ds(jc * TCH, TCH)]],
                         fbuf.at[slot], fsems.at[slot])
        pltpu.async_copy(ctab_hbm.at[ctmp.at[pl.ds(jc * TCH, TCH)]],
                         cbuf.at[slot], csems.at[slot])

    def wait_write(j, slot):
        pltpu.make_async_copy(ftab_hbm.at[ftmp.at[pl.ds(0, TCH)]],
                              fbuf.at[slot], fsems.at[slot]).wait()
        pltpu.make_async_copy(ctab_hbm.at[ctmp.at[pl.ds(0, TCH)]],
                              cbuf.at[slot], csems.at[slot]).wait()
        row0 = tbase + j * TCH
        pltpu.sync_copy(fbuf.at[slot],
                        x_hbm.at[pl.ds(row0, TCH), pl.ds(0, DIM)])
        pltpu.sync_copy(cbuf.at[slot],
                        x_hbm.at[pl.ds(row0, TCH), pl.ds(DIM, DIM)])

    for s in range(NBUF):
        gather(s, s)

    def group(jj, carry):
        j0 = jj * NBUF
        for s in range(NBUF):
            wait_write(j0 + s, s)
            gather(j0 + s + NBUF, s)
        return carry

    lax.fori_loop(0, NCH // NBUF, group, 0)
    for slot in range(NBUF):
        pltpu.make_async_copy(ftab_hbm.at[ftmp.at[pl.ds(0, TCH)]],
                              fbuf.at[slot], fsems.at[slot]).wait()
        pltpu.make_async_copy(ctab_hbm.at[ctmp.at[pl.ds(0, TCH)]],
                              cbuf.at[slot], csems.at[slot]).wait()


def _sc_gather(fidx, cidx, ftab, ctab):
    return pl.kernel(
        _sc_gather_body,
        out_type=jax.ShapeDtypeStruct((NL, 2 * DIM), jnp.float32),
        mesh=plsc.VectorSubcoreMesh(core_axis_name="c", subcore_axis_name="s",
                                    num_cores=NC, num_subcores=NS),
        scratch_types=[
            pltpu.VMEM((PER_W,), jnp.int32),
            pltpu.VMEM((PER_W,), jnp.int32),
            pltpu.VMEM((NBUF, TCH, DIM), jnp.float32),
            pltpu.VMEM((NBUF, TCH, DIM), jnp.float32),
            pltpu.SemaphoreType.DMA((NBUF,)),
            pltpu.SemaphoreType.DMA((NBUF,)),
        ],
        compiler_params=pltpu.CompilerParams(use_tc_tiling_on_sc=False),
    )(fidx, cidx, ftab, ctab)


def _tc_mlp_body(x_ref, w1_ref, b1_ref, w2_ref, b2_ref, o_ref):
    x = x_ref[...]
    h = jnp.dot(x, w1_ref[...], preferred_element_type=jnp.float32)
    h = jnp.maximum(h + b1_ref[...], 0.0)
    y = (jnp.dot(h, w2_ref[...], preferred_element_type=jnp.float32)
         + b2_ref[...])
    o_ref[0] = jnp.transpose(y)


def _tc_mlp(x2, W1, b1, W2, b2, *, interpret=False):
    return pl.pallas_call(
        _tc_mlp_body,
        grid=(L,),
        in_specs=[
            pl.BlockSpec((B, 2 * DIM), lambda i: (i, 0)),
            pl.BlockSpec((2 * DIM, 2 * DIM), lambda i: (0, 0)),
            pl.BlockSpec((1, 2 * DIM), lambda i: (0, 0)),
            pl.BlockSpec((2 * DIM, DIM), lambda i: (0, 0)),
            pl.BlockSpec((1, DIM), lambda i: (0, 0)),
        ],
        out_specs=pl.BlockSpec((1, DIM, B), lambda i: (i, 0, 0)),
        out_shape=jax.ShapeDtypeStruct((L, DIM, B), jnp.float32),
        interpret=interpret,
    )(x2, W1, b1, W2, b2)


def kernel(fine_ids, coarse_ids, fine_table, coarse_table, W1, b1, W2, b2):
    fiT = fine_ids.astype(jnp.int32).T.reshape(NL)
    ciT = coarse_ids.astype(jnp.int32).T.reshape(NL) * 2
    ctab_v = jnp.pad(coarse_table, ((0, 7), (0, DIM))).reshape(2016, DIM)
    x2 = _sc_gather(fiT, ciT, fine_table, ctab_v)
    outT = _tc_mlp(x2, W1, b1.reshape(1, 2 * DIM), W2, b2.reshape(1, DIM))
    return jnp.transpose(outT, (2, 0, 1)), jnp.float32(0.5)


# speedup vs baseline: 2.3026x; 1.0509x over previous
import jax
import jax.numpy as jnp
from jax import lax
from jax.experimental import pallas as pl
from jax.experimental.pallas import tpu as pltpu
from jax.experimental.pallas import tpu_sc as plsc

B, L, DIM = 4096, 50, 64
NL = B * L
NC, NS = 2, 16
NW = NC * NS
PER_W = NL // NW
TCH = 128
NCH = PER_W // TCH
NBUF = 5
FROWS = 100001


def _sc_gather_body(fidx_hbm, cidx_hbm, ftab_hbm, ctab_hbm, x_hbm,
                    ftmp, ctmp, fbuf, cbuf, fsems, csems):
    wid = lax.axis_index("s") * NC + lax.axis_index("c")
    tbase = wid * PER_W
    pltpu.sync_copy(fidx_hbm.at[pl.ds(tbase, PER_W)], ftmp)
    pltpu.sync_copy(cidx_hbm.at[pl.ds(tbase, PER_W)], ctmp)

    def gather(j, slot):
        jc = lax.min(j, NCH - 1)
        pltpu.async_copy(ftab_hbm.at[ftmp.at[pl.ds(jc * TCH, TCH)]],
                         fbuf.at[slot], fsems.at[slot])
        pltpu.async_copy(ctab_hbm.at[ctmp.at[pl.ds(jc * TCH, TCH)]],
                         cbuf.at[slot], csems.at[slot])

    def wait_write(j, slot):
        pltpu.make_async_copy(ftab_hbm.at[ftmp.at[pl.ds(0, TCH)]],
                              fbuf.at[slot], fsems.at[slot]).wait()
        pltpu.make_async_copy(ctab_hbm.at[ctmp.at[pl.ds(0, TCH)]],
                              cbuf.at[slot], csems.at[slot]).wait()
        row0 = tbase + j * TCH
        pltpu.sync_copy(fbuf.at[slot],
                        x_hbm.at[pl.ds(row0, TCH), pl.ds(0, DIM)])
        pltpu.sync_copy(cbuf.at[slot],
                        x_hbm.at[pl.ds(row0, TCH), pl.ds(DIM, DIM)])

    for s in range(NBUF):
        gather(s, s)

    def group(jj, carry):
        j0 = jj * NBUF
        for s in range(NBUF):
            wait_write(j0 + s, s)
            gather(j0 + s + NBUF, s)
        return carry

    lax.fori_loop(0, NCH // NBUF, group, 0)
    for slot in range(NBUF):
        pltpu.make_async_copy(ftab_hbm.at[ftmp.at[pl.ds(0, TCH)]],
                              fbuf.at[slot], fsems.at[slot]).wait()
        pltpu.make_async_copy(ctab_hbm.at[ctmp.at[pl.ds(0, TCH)]],
                              cbuf.at[slot], csems.at[slot]).wait()


def _sc_gather(fidx, cidx, ftab, ctab):
    return pl.kernel(
        _sc_gather_body,
        out_type=jax.ShapeDtypeStruct((NL, 2 * DIM), jnp.float32),
        mesh=plsc.VectorSubcoreMesh(core_axis_name="c", subcore_axis_name="s",
                                    num_cores=NC, num_subcores=NS),
        scratch_types=[
            pltpu.VMEM((PER_W,), jnp.int32),
            pltpu.VMEM((PER_W,), jnp.int32),
            pltpu.VMEM((NBUF, TCH, DIM), jnp.float32),
            pltpu.VMEM((NBUF, TCH, DIM), jnp.float32),
            pltpu.SemaphoreType.DMA((NBUF,)),
            pltpu.SemaphoreType.DMA((NBUF,)),
        ],
        compiler_params=pltpu.CompilerParams(use_tc_tiling_on_sc=False),
    )(fidx, cidx, ftab, ctab)


def _tc_mlp_body(x_ref, w1_ref, b1_ref, w2_ref, b2_ref, o_ref):
    x = x_ref[...]
    h = jnp.dot(x, w1_ref[...], preferred_element_type=jnp.float32)
    h = jnp.maximum(h + b1_ref[...], 0.0)
    y = (jnp.dot(h, w2_ref[...], preferred_element_type=jnp.float32)
         + b2_ref[...])
    o_ref[0] = jnp.transpose(y)


def _tc_mlp(x2, W1, b1, W2, b2, *, interpret=False):
    return pl.pallas_call(
        _tc_mlp_body,
        grid=(L,),
        in_specs=[
            pl.BlockSpec((B, 2 * DIM), lambda i: (i, 0)),
            pl.BlockSpec((2 * DIM, 2 * DIM), lambda i: (0, 0)),
            pl.BlockSpec((1, 2 * DIM), lambda i: (0, 0)),
            pl.BlockSpec((2 * DIM, DIM), lambda i: (0, 0)),
            pl.BlockSpec((1, DIM), lambda i: (0, 0)),
        ],
        out_specs=pl.BlockSpec((1, DIM, B), lambda i: (i, 0, 0)),
        out_shape=jax.ShapeDtypeStruct((L, DIM, B), jnp.float32),
        interpret=interpret,
    )(x2, W1, b1, W2, b2)


def kernel(fine_ids, coarse_ids, fine_table, coarse_table, W1, b1, W2, b2):
    fiT = fine_ids.astype(jnp.int32).T.reshape(NL) * 2
    ciT = coarse_ids.astype(jnp.int32).T.reshape(NL) * 2
    ftab_v = jnp.pad(fine_table, ((0, 7), (0, DIM))).reshape(200016, DIM)
    ctab_v = jnp.pad(coarse_table, ((0, 7), (0, DIM))).reshape(2016, DIM)
    x2 = _sc_gather(fiT, ciT, ftab_v, ctab_v)
    outT = _tc_mlp(x2, W1, b1.reshape(1, 2 * DIM), W2, b2.reshape(1, DIM))
    return jnp.transpose(outT, (2, 0, 1)), jnp.float32(0.5)


# speedup vs baseline: 2.3041x; 1.0007x over previous
import jax
import jax.numpy as jnp
from jax import lax
from jax.experimental import pallas as pl
from jax.experimental.pallas import tpu as pltpu
from jax.experimental.pallas import tpu_sc as plsc

B, L, DIM = 4096, 50, 64
NL = B * L
NC, NS = 2, 16
NW = NC * NS
PER_W = NL // NW
TCH = 128
NCH = PER_W // TCH
NBUF = 5
FROWS = 100001


def _sc_gather_body(fidx_hbm, cidx_hbm, ftab_hbm, ctab_hbm, x_hbm,
                    ftmp, ctmp, fbuf, cbuf, fsems, csems):
    wid = lax.axis_index("s") * NC + lax.axis_index("c")
    tbase = wid * PER_W
    pltpu.sync_copy(fidx_hbm.at[pl.ds(tbase, PER_W)], ftmp)
    pltpu.sync_copy(cidx_hbm.at[pl.ds(tbase, PER_W)], ctmp)

    def gather(j, slot):
        jc = lax.min(j, NCH - 1)
        pltpu.async_copy(ftab_hbm.at[ftmp.at[pl.ds(jc * TCH, TCH)]],
                         fbuf.at[slot], fsems.at[slot])
        pltpu.async_copy(ctab_hbm.at[ctmp.at[pl.ds(jc * TCH, TCH)]],
                         cbuf.at[slot], csems.at[slot])

    def wait_write(j, slot):
        pltpu.make_async_copy(ftab_hbm.at[ftmp.at[pl.ds(0, TCH)]],
                              fbuf.at[slot], fsems.at[slot]).wait()
        pltpu.make_async_copy(ctab_hbm.at[ctmp.at[pl.ds(0, TCH)]],
                              cbuf.at[slot], csems.at[slot]).wait()
        row0 = tbase + j * TCH
        pltpu.sync_copy(fbuf.at[slot],
                        x_hbm.at[pl.ds(row0, TCH), pl.ds(0, DIM)])
        pltpu.sync_copy(cbuf.at[slot],
                        x_hbm.at[pl.ds(row0, TCH), pl.ds(DIM, DIM)])

    for s in range(NBUF):
        gather(s, s)

    def group(jj, carry):
        j0 = jj * NBUF
        for s in range(NBUF):
            wait_write(j0 + s, s)
            gather(j0 + s + NBUF, s)
        return carry

    lax.fori_loop(0, NCH // NBUF, group, 0)
    for slot in range(NBUF):
        pltpu.make_async_copy(ftab_hbm.at[ftmp.at[pl.ds(0, TCH)]],
                              fbuf.at[slot], fsems.at[slot]).wait()
        pltpu.make_async_copy(ctab_hbm.at[ctmp.at[pl.ds(0, TCH)]],
                              cbuf.at[slot], csems.at[slot]).wait()


def _sc_gather(fidx, cidx, ftab, ctab):
    return pl.kernel(
        _sc_gather_body,
        out_type=jax.ShapeDtypeStruct((NL, 2 * DIM), jnp.float32),
        mesh=plsc.VectorSubcoreMesh(core_axis_name="c", subcore_axis_name="s",
                                    num_cores=NC, num_subcores=NS),
        scratch_types=[
            pltpu.VMEM((PER_W,), jnp.int32),
            pltpu.VMEM((PER_W,), jnp.int32),
            pltpu.VMEM((NBUF, TCH, DIM), jnp.float32),
            pltpu.VMEM((NBUF, TCH, DIM), jnp.float32),
            pltpu.SemaphoreType.DMA((NBUF,)),
            pltpu.SemaphoreType.DMA((NBUF,)),
        ],
        compiler_params=pltpu.CompilerParams(use_tc_tiling_on_sc=False),
    )(fidx, cidx, ftab, ctab)


def _tc_mlp_body(x_ref, w1_ref, b1_ref, w2_ref, b2_ref, o_ref):
    x = x_ref[...]
    h = jnp.dot(x, w1_ref[...], preferred_element_type=jnp.float32)
    h = jnp.maximum(h + b1_ref[...], 0.0)
    y = (jnp.dot(h, w2_ref[...], preferred_element_type=jnp.float32)
         + b2_ref[...])
    o_ref[0] = jnp.transpose(y)


def _tc_mlp(x2, W1, b1, W2, b2, *, interpret=False):
    return pl.pallas_call(
        _tc_mlp_body,
        grid=(L,),
        in_specs=[
            pl.BlockSpec((B, 2 * DIM), lambda i: (i, 0)),
            pl.BlockSpec((2 * DIM, 2 * DIM), lambda i: (0, 0)),
            pl.BlockSpec((1, 2 * DIM), lambda i: (0, 0)),
            pl.BlockSpec((2 * DIM, DIM), lambda i: (0, 0)),
            pl.BlockSpec((1, DIM), lambda i: (0, 0)),
        ],
        out_specs=pl.BlockSpec((1, DIM, B), lambda i: (i, 0, 0)),
        out_shape=jax.ShapeDtypeStruct((L, DIM, B), jnp.float32),
        interpret=interpret,
    )(x2, W1, b1, W2, b2)


def kernel(fine_ids, coarse_ids, fine_table, coarse_table, W1, b1, W2, b2):
    fiT = fine_ids.astype(jnp.int32).T.reshape(NL) * 2
    ciT = coarse_ids.astype(jnp.int32).T.reshape(NL) * 2
    ftab_v = (jnp.zeros((100008, 2 * DIM), jnp.float32)
              .at[:FROWS, :DIM].set(fine_table).reshape(200016, DIM))
    ctab_v = jnp.pad(coarse_table, ((0, 7), (0, DIM))).reshape(2016, DIM)
    x2 = _sc_gather(fiT, ciT, ftab_v, ctab_v)
    outT = _tc_mlp(x2, W1, b1.reshape(1, 2 * DIM), W2, b2.reshape(1, DIM))
    return jnp.transpose(outT, (2, 0, 1)), jnp.float32(0.5)
